# Initial kernel scaffold; baseline (speedup 1.0000x reference)
import jax
import jax.numpy as jnp
from jax.experimental import pallas as pl


def kernel(token_embedding, pe, mask_w, ln_gamma, ln_beta, mask_index, unmask_index):
    raise NotImplementedError("write your pallas kernel here")


# speedup vs baseline: 1.9495x; 1.9495x over previous
import functools

import jax
import jax.numpy as jnp
from jax import lax
from jax.experimental import pallas as pl
from jax.experimental.pallas import tpu as pltpu
from jax.experimental.pallas import tpu_sc as plsc


def _ln_table_body(pe_ref, mask_ref, gamma_ref, beta_ref, out_ref):
    pre = pe_ref[...] + mask_ref[...]
    mu = jnp.mean(pre, axis=1, keepdims=True)
    xc = pre - mu
    var = jnp.mean(xc * xc, axis=1, keepdims=True)
    out_ref[...] = xc * lax.rsqrt(var + 1e-5) * gamma_ref[...] + beta_ref[...]


def _ln_table(pe, mask_vec, gamma, beta):
    t, k = pe.shape
    return pl.pallas_call(
        _ln_table_body,
        out_shape=jax.ShapeDtypeStruct((t, k), jnp.float32),
    )(pe, mask_vec, gamma, beta)


_CHUNK = 72


def _make_sc_gather(b, t, k, nm, nu):
    info = plsc.get_sparse_core_info()
    nc, ns = info.num_cores, info.num_subcores
    nw = nc * ns
    assert b % nw == 0
    bpw = b // nw
    assert nu % _CHUNK == 0 or nu < _CHUNK
    assert nm % _CHUNK == 0

    mesh = plsc.VectorSubcoreMesh(core_axis_name="c", subcore_axis_name="s")

    @functools.partial(
        pl.kernel,
        out_type=[
            jax.ShapeDtypeStruct((b * nu, k), jnp.float32),
            jax.ShapeDtypeStruct((b * nm, k), jnp.float32),
            jax.ShapeDtypeStruct((b * nu, k), jnp.float32),
        ],
        mesh=mesh,
        scratch_types=[
            pltpu.VMEM((nu,), jnp.int32),
            pltpu.VMEM((nm,), jnp.int32),
            pltpu.VMEM((nu,), jnp.int32),
            pltpu.VMEM((_CHUNK, k), jnp.float32),
            pltpu.SemaphoreType.DMA,
        ],
    )
    def sc(te_hbm, pe_hbm, ln_hbm, mi_hbm, ui_hbm,
           o_emb, o_mask, o_pos,
           ui_v, mi_v, tei_v, buf, sem):
        wid = lax.axis_index("s") * nc + lax.axis_index("c")

        def gather_rows(tab, idx_ref, n_rows, out, obase):
            for c0 in range(0, n_rows, _CHUNK):
                cc = min(_CHUNK, n_rows - c0)
                cp = pltpu.make_async_copy(
                    tab.at[idx_ref.at[pl.ds(c0, cc)]],
                    buf.at[pl.ds(0, cc)],
                    sem,
                )
                cp.start()
                cp.wait()
                pltpu.sync_copy(buf.at[pl.ds(0, cc)],
                                out.at[pl.ds(obase + c0, cc)])

        for bl in range(bpw):
            bb = wid * bpw + bl
            pltpu.sync_copy(ui_hbm.at[bb], ui_v)
            pltpu.sync_copy(mi_hbm.at[bb], mi_v)
            off = bb * t
            for i in range(nu // 16):
                tei_v[pl.ds(i * 16, 16)] = ui_v[pl.ds(i * 16, 16)] + off
            gather_rows(te_hbm, tei_v, nu, o_emb, bb * nu)
            gather_rows(pe_hbm, ui_v, nu, o_pos, bb * nu)
            gather_rows(ln_hbm, mi_v, nm, o_mask, bb * nm)

    return sc


def kernel(token_embedding, pe, mask_w, ln_gamma, ln_beta,
           mask_index, unmask_index):
    b, t, k = token_embedding.shape
    nm = mask_index.shape[1]
    nu = unmask_index.shape[1]

    mask_vec = jnp.reshape(mask_w, (1, k))
    ln_table = _ln_table(pe, mask_vec,
                         jnp.reshape(ln_gamma, (1, k)),
                         jnp.reshape(ln_beta, (1, k)))

    te_flat = jnp.reshape(token_embedding, (b * t, k))
    o_emb, o_mask, o_pos = _make_sc_gather(b, t, k, nm, nu)(
        te_flat, pe, ln_table, mask_index, unmask_index)

    return (
        jnp.reshape(o_emb, (b, nu, k)),
        jnp.reshape(o_mask, (b, nm, k)),
        jnp.reshape(o_pos, (b, nu, k)),
        mask_index,
        unmask_index,
    )
